# baseline (speedup 1.0000x reference)
import jax
import jax.numpy as jnp
from jax.experimental import pallas as pl
from jax.experimental.pallas import tpu as pltpu

_K = 8
_NACC = 2
_MB = 4
_SSPLIT = 2

_SORT8 = [
    (0, 1), (2, 3), (4, 5), (6, 7),
    (0, 2), (1, 3), (4, 6), (5, 7),
    (1, 2), (5, 6), (0, 4), (3, 7),
    (1, 5), (2, 6),
    (1, 4), (3, 6),
    (2, 4), (3, 5),
    (3, 4),
]

_BITONIC8 = [
    (0, 4), (1, 5), (2, 6), (3, 7),
    (0, 2), (1, 3), (4, 6), (5, 7),
    (0, 1), (2, 3), (4, 5), (6, 7),
]


def _apply_network(w, pairs):
    w = list(w)
    for i, j in pairs:
        hi = jnp.maximum(w[i], w[j])
        lo = jnp.minimum(w[i], w[j])
        w[i] = hi
        w[j] = lo
    return w


def _merge_sorted(acc, new):
    d = [jnp.maximum(acc[i], new[_K - 1 - i]) for i in range(_K)]
    return _apply_network(d, _BITONIC8)


def _topk_one_batch(x_ref, o_ref, acc_ref, fin_ref, bslot, is_first, is_last):
    s = x_ref.shape[1]
    chunk_rows = _K * 8
    iters = s // (chunk_rows * _NACC)

    def body(j, accs):
        out = []
        for a in range(_NACC):
            base = (j * _NACC + a) * chunk_rows
            t = [x_ref[bslot, pl.ds(base + i * 8, 8), :] for i in range(_K)]
            t = _apply_network(t, _SORT8)
            out.append(tuple(_merge_sorted(accs[a], t)))
        return tuple(out)

    neg = jnp.full((8, x_ref.shape[2]), -jnp.inf, dtype=x_ref.dtype)
    accs = tuple(
        tuple(jnp.where(is_first, neg,
                        acc_ref[bslot, pl.ds((a * _K + l) * 8, 8), :])
              for l in range(_K))
        for a in range(_NACC))

    for j in range(iters):
        accs = body(j, accs)

    for a in range(_NACC):
        for l in range(_K):
            acc_ref[bslot, pl.ds((a * _K + l) * 8, 8), :] = accs[a][l]

    @pl.when(is_last)
    def _():
        accl = list(accs)
        while len(accl) > 1:
            accl = [_merge_sorted(accl[i], accl[i + 1])
                    for i in range(0, len(accl), 2)]
        acc = accl[0]

        for l in range(_K):
            fin_ref[pl.ds(8 * l, 8), :] = acc[l]
        w = [fin_ref[pl.ds(8 * i, 8), :] for i in range(_K)]
        half = 4
        while half >= 1:
            top = [w[i][:half, :] for i in range(_K)]
            bot = [w[i][half:, :] for i in range(_K)]
            d = [jnp.maximum(top[i], bot[_K - 1 - i]) for i in range(_K)]
            w = _apply_network(d, _BITONIC8)
            half //= 2

        o_ref[bslot] = jnp.concatenate(w, axis=0)


def _topk_body(x_ref, o_ref, acc_ref, fin_ref):
    jseq = pl.program_id(1)
    is_first = jseq == 0
    is_last = jseq == _SSPLIT - 1
    for bslot in range(_MB):
        _topk_one_batch(x_ref, o_ref, acc_ref, fin_ref, bslot,
                        is_first, is_last)


def kernel(inputs):
    b, s, c = inputs.shape
    out = pl.pallas_call(
        _topk_body,
        grid=(b // _MB, _SSPLIT),
        in_specs=[pl.BlockSpec((_MB, s // _SSPLIT, c),
                               lambda i, j: (i, j, 0))],
        out_specs=pl.BlockSpec((_MB, _K, c), lambda i, j: (i, 0, 0)),
        out_shape=jax.ShapeDtypeStruct((b, _K, c), jnp.float32),
        scratch_shapes=[pltpu.VMEM((_MB, _NACC * _K * 8, c), jnp.float32),
                        pltpu.VMEM((_K * 8, c), jnp.float32)],
    )(inputs)
    return jnp.transpose(out, (0, 2, 1)).reshape(b, c * _K)


# speedup vs baseline: 1.0778x; 1.0778x over previous
import jax
import jax.numpy as jnp
from jax.experimental import pallas as pl
from jax.experimental.pallas import tpu as pltpu

_K = 8

_SORT8 = [
    (0, 1), (2, 3), (4, 5), (6, 7),
    (0, 2), (1, 3), (4, 6), (5, 7),
    (1, 2), (5, 6), (0, 4), (3, 7),
    (1, 5), (2, 6),
    (1, 4), (3, 6),
    (2, 4), (3, 5),
    (3, 4),
]

_BITONIC8 = [
    (0, 4), (1, 5), (2, 6), (3, 7),
    (0, 2), (1, 3), (4, 6), (5, 7),
    (0, 1), (2, 3), (4, 5), (6, 7),
]


def _apply_network(w, pairs):
    w = list(w)
    for i, j in pairs:
        hi = jnp.maximum(w[i], w[j])
        lo = jnp.minimum(w[i], w[j])
        w[i] = hi
        w[j] = lo
    return w


def _merge_sorted(acc, new):
    d = [jnp.maximum(acc[i], new[_K - 1 - i]) for i in range(_K)]
    return _apply_network(d, _BITONIC8)


_NACC = 4


_MB = 4


def _topk_one_batch(x_ref, o_ref, scratch_ref, bslot):
    s = x_ref.shape[1]
    chunk_rows = _K * 8
    iters = s // (chunk_rows * _NACC)

    def body(j, accs):
        out = []
        for a in range(_NACC):
            base = (j * _NACC + a) * chunk_rows
            t = [x_ref[bslot, pl.ds(base + i * 8, 8), :] for i in range(_K)]
            t = _apply_network(t, _SORT8)
            out.append(tuple(_merge_sorted(accs[a], t)))
        return tuple(out)

    neg = jnp.full((8, x_ref.shape[2]), -jnp.inf, dtype=x_ref.dtype)
    accs = ((neg,) * _K,) * _NACC
    for j in range(iters):
        accs = body(j, accs)

    accs = list(accs)
    while len(accs) > 1:
        accs = [_merge_sorted(accs[i], accs[i + 1])
                for i in range(0, len(accs), 2)]
    acc = accs[0]

    for l in range(_K):
        scratch_ref[pl.ds(8 * l, 8), :] = acc[l]
    w = [scratch_ref[pl.ds(8 * i, 8), :] for i in range(_K)]
    half = 4
    while half >= 1:
        top = [w[i][:half, :] for i in range(_K)]
        bot = [w[i][half:, :] for i in range(_K)]
        d = [jnp.maximum(top[i], bot[_K - 1 - i]) for i in range(_K)]
        w = _apply_network(d, _BITONIC8)
        half //= 2

    o_ref[bslot] = jnp.concatenate(w, axis=0)


def _topk_body(x_ref, o_ref, scratch_ref):
    for bslot in range(_MB):
        _topk_one_batch(x_ref, o_ref, scratch_ref, bslot)


def kernel(inputs):
    b, s, c = inputs.shape
    out = pl.pallas_call(
        _topk_body,
        grid=(b // _MB,),
        in_specs=[pl.BlockSpec((_MB, s, c), lambda i: (i, 0, 0))],
        out_specs=pl.BlockSpec((_MB, _K, c), lambda i: (i, 0, 0)),
        out_shape=jax.ShapeDtypeStruct((b, _K, c), jnp.float32),
        scratch_shapes=[pltpu.VMEM((_K * 8, c), jnp.float32)],
    )(inputs)
    return jnp.transpose(out, (0, 2, 1)).reshape(b, c * _K)


# speedup vs baseline: 1.0794x; 1.0015x over previous
import jax
import jax.numpy as jnp
from jax.experimental import pallas as pl
from jax.experimental.pallas import tpu as pltpu

_K = 8

_SORT8 = [
    (0, 1), (2, 3), (4, 5), (6, 7),
    (0, 2), (1, 3), (4, 6), (5, 7),
    (1, 2), (5, 6), (0, 4), (3, 7),
    (1, 5), (2, 6),
    (1, 4), (3, 6),
    (2, 4), (3, 5),
    (3, 4),
]

_BITONIC8 = [
    (0, 4), (1, 5), (2, 6), (3, 7),
    (0, 2), (1, 3), (4, 6), (5, 7),
    (0, 1), (2, 3), (4, 5), (6, 7),
]


def _apply_network(w, pairs):
    w = list(w)
    for i, j in pairs:
        hi = jnp.maximum(w[i], w[j])
        lo = jnp.minimum(w[i], w[j])
        w[i] = hi
        w[j] = lo
    return w


def _merge_sorted(acc, new):
    d = [jnp.maximum(acc[i], new[_K - 1 - i]) for i in range(_K)]
    return _apply_network(d, _BITONIC8)


_NACC = 2


_MB = 4


def _topk_one_batch(x_ref, o_ref, scratch_ref, bslot):
    s = x_ref.shape[1]
    chunk_rows = _K * 8
    iters = s // (chunk_rows * _NACC)

    def body(j, accs):
        out = []
        for a in range(_NACC):
            base = (j * _NACC + a) * chunk_rows
            t = [x_ref[bslot, pl.ds(base + i * 8, 8), :] for i in range(_K)]
            t = _apply_network(t, _SORT8)
            out.append(tuple(_merge_sorted(accs[a], t)))
        return tuple(out)

    neg = jnp.full((8, x_ref.shape[2]), -jnp.inf, dtype=x_ref.dtype)
    accs = ((neg,) * _K,) * _NACC
    for j in range(iters):
        accs = body(j, accs)

    accs = list(accs)
    while len(accs) > 1:
        accs = [_merge_sorted(accs[i], accs[i + 1])
                for i in range(0, len(accs), 2)]
    acc = accs[0]

    for l in range(_K):
        scratch_ref[pl.ds(8 * l, 8), :] = acc[l]
    w = [scratch_ref[pl.ds(8 * i, 8), :] for i in range(_K)]
    half = 4
    while half >= 1:
        top = [w[i][:half, :] for i in range(_K)]
        bot = [w[i][half:, :] for i in range(_K)]
        d = [jnp.maximum(top[i], bot[_K - 1 - i]) for i in range(_K)]
        w = _apply_network(d, _BITONIC8)
        half //= 2

    o_ref[bslot] = jnp.concatenate(w, axis=0)


def _topk_body(x_ref, o_ref, scratch_ref):
    for bslot in range(_MB):
        _topk_one_batch(x_ref, o_ref, scratch_ref, bslot)


def kernel(inputs):
    b, s, c = inputs.shape
    out = pl.pallas_call(
        _topk_body,
        grid=(b // _MB,),
        in_specs=[pl.BlockSpec((_MB, s, c), lambda i: (i, 0, 0))],
        out_specs=pl.BlockSpec((_MB, _K, c), lambda i: (i, 0, 0)),
        out_shape=jax.ShapeDtypeStruct((b, _K, c), jnp.float32),
        scratch_shapes=[pltpu.VMEM((_K * 8, c), jnp.float32)],
    )(inputs)
    return jnp.transpose(out, (0, 2, 1)).reshape(b, c * _K)
